# baseline (speedup 1.0000x reference)
import functools

import jax
import jax.numpy as jnp
from jax import lax
from jax.experimental import pallas as pl
from jax.experimental.pallas import tpu as pltpu
from jax.experimental.pallas import tpu_sc as plsc

N = 50000
E = 800000
NG = 128
NC = 2
NS = 16
ZCH = 2000
NZ = N // ZCH
EB = 128
EP = 819200
ER = EP // EB
RPB = 5
PAD = 2048
FC = 32
BN = 2000
NB = N // BN

_MESH = plsc.VectorSubcoreMesh(
    core_axis_name="c", subcore_axis_name="s", num_cores=NC, num_subcores=NS)
_SC_PARAMS = pltpu.CompilerParams(use_tc_tiling_on_sc=False)


def _zero_acc(zeros_hbm, acc, sid):
    for k in range(2):
        ch = sid + NS * k

        @pl.when(ch < NZ)
        def _():
            pltpu.sync_copy(zeros_hbm, acc.at[pl.ds(ch * ZCH, ZCH)])


def _write_out(acc, out_hbm, sid, slot):
    for k in range(2):
        ch = sid + NS * k

        @pl.when(ch < NZ)
        def _():
            pltpu.sync_copy(acc.at[pl.ds(ch * ZCH, ZCH)],
                            out_hbm.at[pl.ds(slot * N + ch * ZCH, ZCH)])


def _deg_body(dst_hbm, ones_hbm, zeros_hbm, out_hbm, idx_d, ones_v, acc, ssem):
    cid = lax.axis_index("c")
    sid = lax.axis_index("s")
    _zero_acc(zeros_hbm, acc, sid)
    pltpu.sync_copy(ones_hbm, ones_v)
    plsc.subcore_barrier()
    rows_sub = (ER // NC) // NS
    base = cid * (ER // NC) + sid * rows_sub

    def step(b, carry):
        pltpu.sync_copy(dst_hbm.at[pl.ds(base + b * RPB, RPB)], idx_d)
        descs = [pltpu.async_copy(ones_v, acc.at[idx_d.at[j]], ssem, add=True)
                 for j in range(RPB)]
        for d in descs:
            d.wait()
        return carry

    lax.fori_loop(0, rows_sub // RPB, step, 0)
    plsc.subcore_barrier()
    _write_out(acc, out_hbm, sid, cid)


_deg_call = pl.kernel(
    _deg_body,
    out_type=jax.ShapeDtypeStruct((NC * N, 8), jnp.float32),
    mesh=_MESH,
    compiler_params=_SC_PARAMS,
    scratch_types=[
        pltpu.VMEM((RPB, EB), jnp.int32),
        pltpu.VMEM((EB, 8), jnp.float32),
        pltpu.VMEM_SHARED((N + PAD, 8), jnp.float32),
        pltpu.SemaphoreType.DMA,
    ],
)


def _make_agg(n_chunks):
    split_edges = n_chunks == 1
    tasks = max(n_chunks // NC, 1)
    rows_core = ER // NC if split_edges else ER
    rows_sub = rows_core // NS
    out_slots = max(n_chunks, 2)

    def body(srcoff_hbm, dst_hbm, table_hbm, zeros_hbm, out_hbm,
             idx_s, idx_d, rows_v, acc, gsem, ssem):
        cid = lax.axis_index("c")
        sid = lax.axis_index("s")
        for k in range(tasks):
            if split_edges:
                chunk = 0
                ebase = cid * rows_core
                slot = cid
            else:
                chunk = cid * tasks + k
                ebase = 0
                slot = chunk
            _zero_acc(zeros_hbm, acc, sid)
            plsc.subcore_barrier()
            base = chunk * ER + ebase + sid * rows_sub
            dbase = ebase + sid * rows_sub

            def step(b, carry):
                pltpu.sync_copy(srcoff_hbm.at[pl.ds(base + b * RPB, RPB)], idx_s)
                gd = [pltpu.async_copy(table_hbm.at[idx_s.at[j]],
                                       rows_v.at[pl.ds(j * EB, EB)], gsem)
                      for j in range(RPB)]
                pltpu.sync_copy(dst_hbm.at[pl.ds(dbase + b * RPB, RPB)], idx_d)
                sd = []
                for j in range(RPB):
                    gd[j].wait()
                    sd.append(pltpu.async_copy(rows_v.at[pl.ds(j * EB, EB)],
                                               acc.at[idx_d.at[j]], ssem,
                                               add=True))
                for d in sd:
                    d.wait()
                return carry

            lax.fori_loop(0, rows_sub // RPB, step, 0)
            plsc.subcore_barrier()
            _write_out(acc, out_hbm, sid, slot)
            plsc.subcore_barrier()

    return pl.kernel(
        body,
        out_type=jax.ShapeDtypeStruct((out_slots * N, FC), jnp.float32),
        mesh=_MESH,
        compiler_params=_SC_PARAMS,
        scratch_types=[
            pltpu.VMEM((RPB, EB), jnp.int32),
            pltpu.VMEM((RPB, EB), jnp.int32),
            pltpu.VMEM((RPB * EB, FC), jnp.float32),
            pltpu.VMEM_SHARED((N + PAD, FC), jnp.float32),
            pltpu.SemaphoreType.DMA,
            pltpu.SemaphoreType.DMA,
        ],
    )


_agg1 = _make_agg(1)
_agg2 = _make_agg(2)
_agg3 = _make_agg(4)


def _prep_body(degp_ref, x_ref, w1_ref, dinv_ref, hs1_ref):
    deg = degp_ref[0, :, 0:1] + degp_ref[1, :, 0:1] + 1.0
    dinv = lax.rsqrt(deg)
    h = jnp.dot(x_ref[...], w1_ref[...], preferred_element_type=jnp.float32)
    dinv_ref[...] = dinv
    hs1_ref[0] = dinv * h


_prep_call = pl.pallas_call(
    _prep_body,
    grid=(NB,),
    in_specs=[
        pl.BlockSpec((NC, BN, 8), lambda i: (0, i, 0)),
        pl.BlockSpec((BN, 3), lambda i: (i, 0)),
        pl.BlockSpec((3, FC), lambda i: (0, 0)),
    ],
    out_specs=[
        pl.BlockSpec((BN, 1), lambda i: (i, 0)),
        pl.BlockSpec((1, BN, FC), lambda i: (0, i, 0)),
    ],
    out_shape=[
        jax.ShapeDtypeStruct((N, 1), jnp.float32),
        jax.ShapeDtypeStruct((1, N, FC), jnp.float32),
    ],
)


def _layer_t(agg_ref, hs_ref, dinv_ref, b_ref, c, partial):
    a = (agg_ref[0] + agg_ref[1]) if partial else agg_ref[c]
    return dinv_ref[...] * (a + hs_ref[c]) + b_ref[c:c + 1, :]


def _norm_relu(t, s1_acc, s2_acc, gw_ref, gb_ref, ga_ref, c):
    mu = s1_acc[c:c + 1, :] * (1.0 / N)
    m2 = s2_acc[c:c + 1, :] * (1.0 / N)
    al = ga_ref[c:c + 1, :]
    var = m2 - (2.0 * al - al * al) * mu * mu
    u = t - al * mu
    y = gw_ref[c:c + 1, :] * u * lax.rsqrt(var + 1e-5) + gb_ref[c:c + 1, :]
    return jnp.maximum(y, 0.0)


def _stats_phase(agg_ref, hs_ref, dinv_ref, b_ref, s1_acc, s2_acc, C, partial, i):
    @pl.when(i == 0)
    def _():
        s1_acc[...] = jnp.zeros_like(s1_acc)
        s2_acc[...] = jnp.zeros_like(s2_acc)

    for c in range(C):
        t = _layer_t(agg_ref, hs_ref, dinv_ref, b_ref, c, partial)
        s1_acc[c:c + 1, :] += jnp.sum(t, axis=0, keepdims=True)
        s2_acc[c:c + 1, :] += jnp.sum(t * t, axis=0, keepdims=True)


def _make_layer(S, C, partial, C_out):
    def body(agg_ref, hs_ref, dinv_ref, b_ref, gw_ref, gb_ref, ga_ref,
             wn_ref, hsn_ref, s1_acc, s2_acc):
        p = pl.program_id(0)
        i = pl.program_id(1)

        @pl.when(p == 0)
        def _():
            _stats_phase(agg_ref, hs_ref, dinv_ref, b_ref, s1_acc, s2_acc,
                         C, partial, i)

        @pl.when(p == 1)
        def _():
            hn = jnp.zeros((BN, FC * C_out), jnp.float32)
            for c in range(C):
                t = _layer_t(agg_ref, hs_ref, dinv_ref, b_ref, c, partial)
                y = _norm_relu(t, s1_acc, s2_acc, gw_ref, gb_ref, ga_ref, c)
                hn = hn + jnp.dot(y, wn_ref[c * FC:(c + 1) * FC, :],
                                  preferred_element_type=jnp.float32)
            hsn = dinv_ref[...] * hn
            for k in range(C_out):
                hsn_ref[k] = hsn[:, k * FC:(k + 1) * FC]

    return pl.pallas_call(
        body,
        grid=(2, NB),
        in_specs=[
            pl.BlockSpec((S, BN, FC), lambda p, i: (0, i, 0)),
            pl.BlockSpec((C, BN, FC), lambda p, i: (0, i, 0)),
            pl.BlockSpec((BN, 1), lambda p, i: (i, 0)),
            pl.BlockSpec((C, FC), lambda p, i: (0, 0)),
            pl.BlockSpec((C, FC), lambda p, i: (0, 0)),
            pl.BlockSpec((C, FC), lambda p, i: (0, 0)),
            pl.BlockSpec((C, FC), lambda p, i: (0, 0)),
            pl.BlockSpec((C * FC, C_out * FC), lambda p, i: (0, 0)),
        ],
        out_specs=pl.BlockSpec((C_out, BN, FC), lambda p, i: (0, i, 0)),
        out_shape=jax.ShapeDtypeStruct((C_out, N, FC), jnp.float32),
        scratch_shapes=[
            pltpu.VMEM((C, FC), jnp.float32),
            pltpu.VMEM((C, FC), jnp.float32),
        ],
    )


def _pool_body(agg_ref, hs_ref, dinv_ref, b_ref, gw_ref, gb_ref, ga_ref,
               batch_ref, linw_ref, linb_ref, out_ref,
               s1_acc, s2_acc, pool_acc, cnt_acc):
    p = pl.program_id(0)
    i = pl.program_id(1)

    @pl.when(p == 0)
    def _():
        _stats_phase(agg_ref, hs_ref, dinv_ref, b_ref, s1_acc, s2_acc,
                     4, False, i)

    @pl.when(p == 1)
    def _():
        @pl.when(i == 0)
        def _():
            pool_acc[...] = jnp.zeros_like(pool_acc)
            cnt_acc[...] = jnp.zeros_like(cnt_acc)

        gid = lax.broadcasted_iota(jnp.int32, (NG, 1), 0)
        mask = (gid == batch_ref[0]).astype(jnp.float32)
        cnt_acc[...] += jnp.sum(mask, axis=1, keepdims=True)
        for c in range(4):
            t = _layer_t(agg_ref, hs_ref, dinv_ref, b_ref, c, False)
            y = _norm_relu(t, s1_acc, s2_acc, gw_ref, gb_ref, ga_ref, c)
            pool_acc[:, c * FC:(c + 1) * FC] += jnp.dot(
                mask, y, preferred_element_type=jnp.float32)

        @pl.when(i == NB - 1)
        def _():
            pooled = pool_acc[...] / jnp.maximum(cnt_acc[...], 1.0)
            out_ref[...] = jnp.dot(pooled, linw_ref[...],
                                   preferred_element_type=jnp.float32) + linb_ref[...]


_pool_call = pl.pallas_call(
    _pool_body,
    grid=(2, NB),
    in_specs=[
        pl.BlockSpec((4, BN, FC), lambda p, i: (0, i, 0)),
        pl.BlockSpec((4, BN, FC), lambda p, i: (0, i, 0)),
        pl.BlockSpec((BN, 1), lambda p, i: (i, 0)),
        pl.BlockSpec((4, FC), lambda p, i: (0, 0)),
        pl.BlockSpec((4, FC), lambda p, i: (0, 0)),
        pl.BlockSpec((4, FC), lambda p, i: (0, 0)),
        pl.BlockSpec((4, FC), lambda p, i: (0, 0)),
        pl.BlockSpec((1, 1, BN), lambda p, i: (i, 0, 0)),
        pl.BlockSpec((4 * FC, 3), lambda p, i: (0, 0)),
        pl.BlockSpec((1, 3), lambda p, i: (0, 0)),
    ],
    out_specs=pl.BlockSpec((NG, 3), lambda p, i: (0, 0)),
    out_shape=jax.ShapeDtypeStruct((NG, 3), jnp.float32),
    scratch_shapes=[
        pltpu.VMEM((4, FC), jnp.float32),
        pltpu.VMEM((4, FC), jnp.float32),
        pltpu.VMEM((NG, 4 * FC), jnp.float32),
        pltpu.VMEM((NG, 1), jnp.float32),
    ],
)

_layer1 = _make_layer(2, 1, True, 2)
_layer2 = _make_layer(2, 2, False, 4)


def kernel(x, edge_index, batch, W1, b1, g1w, g1b, g1a, W2, b2, g2w, g2b, g2a,
           W3, b3, g3w, g3b, g3a, linW, linb):
    pad = EP - E
    src = jnp.concatenate(
        [edge_index[0], jnp.arange(pad, dtype=jnp.int32) % 1024])
    dst2d = jnp.concatenate(
        [edge_index[1],
         N + (jnp.arange(pad, dtype=jnp.int32) % PAD)]).reshape(ER, EB)
    srcoff = (src[None, :]
              + (jnp.arange(4, dtype=jnp.int32) * N)[:, None]).reshape(4 * ER, EB)
    zeros_fc = jnp.zeros((ZCH, FC), jnp.float32)
    zeros_8 = jnp.zeros((ZCH, 8), jnp.float32)
    ones_8 = jnp.ones((EB, 8), jnp.float32)

    degp = _deg_call(dst2d, ones_8, zeros_8).reshape(NC, N, 8)
    dinv, hs1 = _prep_call(degp, x, W1)

    agg1 = _agg1(srcoff, dst2d, hs1.reshape(N, FC), zeros_fc).reshape(2, N, FC)
    hs2 = _layer1(agg1, hs1, dinv, b1.reshape(1, FC),
                  g1w.reshape(1, FC), g1b.reshape(1, FC), g1a.reshape(1, FC), W2)

    agg2 = _agg2(srcoff, dst2d, hs2.reshape(2 * N, FC),
                 zeros_fc).reshape(2, N, FC)
    hs3 = _layer2(agg2, hs2, dinv, b2.reshape(2, FC),
                  g2w.reshape(2, FC), g2b.reshape(2, FC), g2a.reshape(2, FC), W3)

    agg3 = _agg3(srcoff, dst2d, hs3.reshape(4 * N, FC),
                 zeros_fc).reshape(4, N, FC)
    out = _pool_call(agg3, hs3, dinv, b3.reshape(4, FC),
                     g3w.reshape(4, FC), g3b.reshape(4, FC), g3a.reshape(4, FC),
                     batch.reshape(NB, 1, BN), linW, linb.reshape(1, 3))
    return out


# speedup vs baseline: 1.0479x; 1.0479x over previous
import functools

import jax
import jax.numpy as jnp
from jax import lax
from jax.experimental import pallas as pl
from jax.experimental.pallas import tpu as pltpu
from jax.experimental.pallas import tpu_sc as plsc

N = 50000
E = 800000
NG = 128
NC = 2
NS = 16
ZCH = 2000
NZ = N // ZCH
EB = 128
EP = 819200
ER = EP // EB
RPB = 5
PAD = 2048
FC = 32
BN = 2000
NB = N // BN

_MESH = plsc.VectorSubcoreMesh(
    core_axis_name="c", subcore_axis_name="s", num_cores=NC, num_subcores=NS)
_SC_PARAMS = pltpu.CompilerParams(use_tc_tiling_on_sc=False)


def _zero_acc(zeros_hbm, acc, sid):
    for k in range(2):
        ch = sid + NS * k

        @pl.when(ch < NZ)
        def _():
            pltpu.sync_copy(zeros_hbm, acc.at[pl.ds(ch * ZCH, ZCH)])


def _write_out(acc, out_hbm, sid, slot):
    for k in range(2):
        ch = sid + NS * k

        @pl.when(ch < NZ)
        def _():
            pltpu.sync_copy(acc.at[pl.ds(ch * ZCH, ZCH)],
                            out_hbm.at[pl.ds(slot * N + ch * ZCH, ZCH)])


def _deg_body(dst_hbm, ones_hbm, zeros_hbm, out_hbm, idx_d, ones_v, acc, ssem):
    cid = lax.axis_index("c")
    sid = lax.axis_index("s")
    _zero_acc(zeros_hbm, acc, sid)
    pltpu.sync_copy(ones_hbm, ones_v)
    plsc.subcore_barrier()
    rows_sub = (ER // NC) // NS
    base = cid * (ER // NC) + sid * rows_sub

    def step(b, carry):
        pltpu.sync_copy(dst_hbm.at[pl.ds(base + b * RPB, RPB)], idx_d)
        descs = [pltpu.async_copy(ones_v, acc.at[idx_d.at[j]], ssem, add=True)
                 for j in range(RPB)]
        for d in descs:
            d.wait()
        return carry

    lax.fori_loop(0, rows_sub // RPB, step, 0)
    plsc.subcore_barrier()
    _write_out(acc, out_hbm, sid, cid)


_deg_call = pl.kernel(
    _deg_body,
    out_type=jax.ShapeDtypeStruct((NC * N, 8), jnp.float32),
    mesh=_MESH,
    compiler_params=_SC_PARAMS,
    scratch_types=[
        pltpu.VMEM((RPB, EB), jnp.int32),
        pltpu.VMEM((EB, 8), jnp.float32),
        pltpu.VMEM_SHARED((N + PAD, 8), jnp.float32),
        pltpu.SemaphoreType.DMA,
    ],
)


def _make_agg(n_chunks):
    split_edges = n_chunks == 1
    tasks = max(n_chunks // NC, 1)
    rows_core = ER // NC if split_edges else ER
    rows_sub = rows_core // NS
    out_slots = max(n_chunks, 2)

    def body(srcoff_hbm, dst_hbm, table_hbm, zeros_hbm, out_hbm,
             idx_s, idx_d, rows_v, acc, gsem, ssem):
        cid = lax.axis_index("c")
        sid = lax.axis_index("s")
        for k in range(tasks):
            if split_edges:
                chunk = 0
                ebase = cid * rows_core
                slot = cid
            else:
                chunk = cid * tasks + k
                ebase = 0
                slot = chunk
            _zero_acc(zeros_hbm, acc, sid)
            plsc.subcore_barrier()
            base = chunk * ER + ebase + sid * rows_sub
            dbase = ebase + sid * rows_sub

            def step(b, carry):
                pltpu.sync_copy(srcoff_hbm.at[pl.ds(base + b * RPB, RPB)], idx_s)
                gd = []
                for j in range(RPB):
                    @pl.when(b > 0)
                    def _(j=j):
                        pltpu.make_async_copy(
                            table_hbm.at[pl.ds(0, EB)],
                            rows_v.at[pl.ds(j * EB, EB)], ssem).wait()

                    gd.append(pltpu.async_copy(table_hbm.at[idx_s.at[j]],
                                               rows_v.at[pl.ds(j * EB, EB)],
                                               gsem))
                pltpu.sync_copy(dst_hbm.at[pl.ds(dbase + b * RPB, RPB)], idx_d)
                for j in range(RPB):
                    gd[j].wait()
                    pltpu.async_copy(rows_v.at[pl.ds(j * EB, EB)],
                                     acc.at[idx_d.at[j]], ssem, add=True)
                return carry

            lax.fori_loop(0, rows_sub // RPB, step, 0)
            for j in range(RPB):
                pltpu.make_async_copy(table_hbm.at[pl.ds(0, EB)],
                                      rows_v.at[pl.ds(j * EB, EB)], ssem).wait()
            plsc.subcore_barrier()
            _write_out(acc, out_hbm, sid, slot)
            plsc.subcore_barrier()

    return pl.kernel(
        body,
        out_type=jax.ShapeDtypeStruct((out_slots * N, FC), jnp.float32),
        mesh=_MESH,
        compiler_params=_SC_PARAMS,
        scratch_types=[
            pltpu.VMEM((RPB, EB), jnp.int32),
            pltpu.VMEM((RPB, EB), jnp.int32),
            pltpu.VMEM((RPB * EB, FC), jnp.float32),
            pltpu.VMEM_SHARED((N + PAD, FC), jnp.float32),
            pltpu.SemaphoreType.DMA,
            pltpu.SemaphoreType.DMA,
        ],
    )


_agg1 = _make_agg(1)
_agg2 = _make_agg(2)
_agg3 = _make_agg(4)


def _prep_body(degp_ref, x_ref, w1_ref, dinv_ref, hs1_ref):
    deg = degp_ref[0, :, 0:1] + degp_ref[1, :, 0:1] + 1.0
    dinv = lax.rsqrt(deg)
    h = jnp.dot(x_ref[...], w1_ref[...], preferred_element_type=jnp.float32)
    dinv_ref[...] = dinv
    hs1_ref[0] = dinv * h


_prep_call = pl.pallas_call(
    _prep_body,
    grid=(NB,),
    in_specs=[
        pl.BlockSpec((NC, BN, 8), lambda i: (0, i, 0)),
        pl.BlockSpec((BN, 3), lambda i: (i, 0)),
        pl.BlockSpec((3, FC), lambda i: (0, 0)),
    ],
    out_specs=[
        pl.BlockSpec((BN, 1), lambda i: (i, 0)),
        pl.BlockSpec((1, BN, FC), lambda i: (0, i, 0)),
    ],
    out_shape=[
        jax.ShapeDtypeStruct((N, 1), jnp.float32),
        jax.ShapeDtypeStruct((1, N, FC), jnp.float32),
    ],
)


def _layer_t(agg_ref, hs_ref, dinv_ref, b_ref, c, partial):
    a = (agg_ref[0] + agg_ref[1]) if partial else agg_ref[c]
    return dinv_ref[...] * (a + hs_ref[c]) + b_ref[c:c + 1, :]


def _norm_relu(t, s1_acc, s2_acc, gw_ref, gb_ref, ga_ref, c):
    mu = s1_acc[c:c + 1, :] * (1.0 / N)
    m2 = s2_acc[c:c + 1, :] * (1.0 / N)
    al = ga_ref[c:c + 1, :]
    var = m2 - (2.0 * al - al * al) * mu * mu
    u = t - al * mu
    y = gw_ref[c:c + 1, :] * u * lax.rsqrt(var + 1e-5) + gb_ref[c:c + 1, :]
    return jnp.maximum(y, 0.0)


def _stats_phase(agg_ref, hs_ref, dinv_ref, b_ref, s1_acc, s2_acc, C, partial, i):
    @pl.when(i == 0)
    def _():
        s1_acc[...] = jnp.zeros_like(s1_acc)
        s2_acc[...] = jnp.zeros_like(s2_acc)

    for c in range(C):
        t = _layer_t(agg_ref, hs_ref, dinv_ref, b_ref, c, partial)
        s1_acc[c:c + 1, :] += jnp.sum(t, axis=0, keepdims=True)
        s2_acc[c:c + 1, :] += jnp.sum(t * t, axis=0, keepdims=True)


def _make_layer(S, C, partial, C_out):
    def body(agg_ref, hs_ref, dinv_ref, b_ref, gw_ref, gb_ref, ga_ref,
             wn_ref, hsn_ref, s1_acc, s2_acc):
        p = pl.program_id(0)
        i = pl.program_id(1)

        @pl.when(p == 0)
        def _():
            _stats_phase(agg_ref, hs_ref, dinv_ref, b_ref, s1_acc, s2_acc,
                         C, partial, i)

        @pl.when(p == 1)
        def _():
            hn = jnp.zeros((BN, FC * C_out), jnp.float32)
            for c in range(C):
                t = _layer_t(agg_ref, hs_ref, dinv_ref, b_ref, c, partial)
                y = _norm_relu(t, s1_acc, s2_acc, gw_ref, gb_ref, ga_ref, c)
                hn = hn + jnp.dot(y, wn_ref[c * FC:(c + 1) * FC, :],
                                  preferred_element_type=jnp.float32)
            hsn = dinv_ref[...] * hn
            for k in range(C_out):
                hsn_ref[k] = hsn[:, k * FC:(k + 1) * FC]

    return pl.pallas_call(
        body,
        grid=(2, NB),
        in_specs=[
            pl.BlockSpec((S, BN, FC), lambda p, i: (0, i, 0)),
            pl.BlockSpec((C, BN, FC), lambda p, i: (0, i, 0)),
            pl.BlockSpec((BN, 1), lambda p, i: (i, 0)),
            pl.BlockSpec((C, FC), lambda p, i: (0, 0)),
            pl.BlockSpec((C, FC), lambda p, i: (0, 0)),
            pl.BlockSpec((C, FC), lambda p, i: (0, 0)),
            pl.BlockSpec((C, FC), lambda p, i: (0, 0)),
            pl.BlockSpec((C * FC, C_out * FC), lambda p, i: (0, 0)),
        ],
        out_specs=pl.BlockSpec((C_out, BN, FC), lambda p, i: (0, i, 0)),
        out_shape=jax.ShapeDtypeStruct((C_out, N, FC), jnp.float32),
        scratch_shapes=[
            pltpu.VMEM((C, FC), jnp.float32),
            pltpu.VMEM((C, FC), jnp.float32),
        ],
    )


def _pool_body(agg_ref, hs_ref, dinv_ref, b_ref, gw_ref, gb_ref, ga_ref,
               batch_ref, linw_ref, linb_ref, out_ref,
               s1_acc, s2_acc, pool_acc, cnt_acc):
    p = pl.program_id(0)
    i = pl.program_id(1)

    @pl.when(p == 0)
    def _():
        _stats_phase(agg_ref, hs_ref, dinv_ref, b_ref, s1_acc, s2_acc,
                     4, False, i)

    @pl.when(p == 1)
    def _():
        @pl.when(i == 0)
        def _():
            pool_acc[...] = jnp.zeros_like(pool_acc)
            cnt_acc[...] = jnp.zeros_like(cnt_acc)

        gid = lax.broadcasted_iota(jnp.int32, (NG, 1), 0)
        mask = (gid == batch_ref[0]).astype(jnp.float32)
        cnt_acc[...] += jnp.sum(mask, axis=1, keepdims=True)
        for c in range(4):
            t = _layer_t(agg_ref, hs_ref, dinv_ref, b_ref, c, False)
            y = _norm_relu(t, s1_acc, s2_acc, gw_ref, gb_ref, ga_ref, c)
            pool_acc[:, c * FC:(c + 1) * FC] += jnp.dot(
                mask, y, preferred_element_type=jnp.float32)

        @pl.when(i == NB - 1)
        def _():
            pooled = pool_acc[...] / jnp.maximum(cnt_acc[...], 1.0)
            out_ref[...] = jnp.dot(pooled, linw_ref[...],
                                   preferred_element_type=jnp.float32) + linb_ref[...]


_pool_call = pl.pallas_call(
    _pool_body,
    grid=(2, NB),
    in_specs=[
        pl.BlockSpec((4, BN, FC), lambda p, i: (0, i, 0)),
        pl.BlockSpec((4, BN, FC), lambda p, i: (0, i, 0)),
        pl.BlockSpec((BN, 1), lambda p, i: (i, 0)),
        pl.BlockSpec((4, FC), lambda p, i: (0, 0)),
        pl.BlockSpec((4, FC), lambda p, i: (0, 0)),
        pl.BlockSpec((4, FC), lambda p, i: (0, 0)),
        pl.BlockSpec((4, FC), lambda p, i: (0, 0)),
        pl.BlockSpec((1, 1, BN), lambda p, i: (i, 0, 0)),
        pl.BlockSpec((4 * FC, 3), lambda p, i: (0, 0)),
        pl.BlockSpec((1, 3), lambda p, i: (0, 0)),
    ],
    out_specs=pl.BlockSpec((NG, 3), lambda p, i: (0, 0)),
    out_shape=jax.ShapeDtypeStruct((NG, 3), jnp.float32),
    scratch_shapes=[
        pltpu.VMEM((4, FC), jnp.float32),
        pltpu.VMEM((4, FC), jnp.float32),
        pltpu.VMEM((NG, 4 * FC), jnp.float32),
        pltpu.VMEM((NG, 1), jnp.float32),
    ],
)

_layer1 = _make_layer(2, 1, True, 2)
_layer2 = _make_layer(2, 2, False, 4)


def kernel(x, edge_index, batch, W1, b1, g1w, g1b, g1a, W2, b2, g2w, g2b, g2a,
           W3, b3, g3w, g3b, g3a, linW, linb):
    pad = EP - E
    src = jnp.concatenate(
        [edge_index[0], jnp.arange(pad, dtype=jnp.int32) % 1024])
    dst2d = jnp.concatenate(
        [edge_index[1],
         N + (jnp.arange(pad, dtype=jnp.int32) % PAD)]).reshape(ER, EB)
    srcoff = (src[None, :]
              + (jnp.arange(4, dtype=jnp.int32) * N)[:, None]).reshape(4 * ER, EB)
    zeros_fc = jnp.zeros((ZCH, FC), jnp.float32)
    zeros_8 = jnp.zeros((ZCH, 8), jnp.float32)
    ones_8 = jnp.ones((EB, 8), jnp.float32)

    degp = _deg_call(dst2d, ones_8, zeros_8).reshape(NC, N, 8)
    dinv, hs1 = _prep_call(degp, x, W1)

    agg1 = _agg1(srcoff, dst2d, hs1.reshape(N, FC), zeros_fc).reshape(2, N, FC)
    hs2 = _layer1(agg1, hs1, dinv, b1.reshape(1, FC),
                  g1w.reshape(1, FC), g1b.reshape(1, FC), g1a.reshape(1, FC), W2)

    agg2 = _agg2(srcoff, dst2d, hs2.reshape(2 * N, FC),
                 zeros_fc).reshape(2, N, FC)
    hs3 = _layer2(agg2, hs2, dinv, b2.reshape(2, FC),
                  g2w.reshape(2, FC), g2b.reshape(2, FC), g2a.reshape(2, FC), W3)

    agg3 = _agg3(srcoff, dst2d, hs3.reshape(4 * N, FC),
                 zeros_fc).reshape(4, N, FC)
    out = _pool_call(agg3, hs3, dinv, b3.reshape(4, FC),
                     g3w.reshape(4, FC), g3b.reshape(4, FC), g3a.reshape(4, FC),
                     batch.reshape(NB, 1, BN), linW, linb.reshape(1, 3))
    return out


# speedup vs baseline: 1.0836x; 1.0341x over previous
import functools

import jax
import jax.numpy as jnp
from jax import lax
from jax.experimental import pallas as pl
from jax.experimental.pallas import tpu as pltpu
from jax.experimental.pallas import tpu_sc as plsc

N = 50000
E = 800000
NG = 128
NC = 2
NS = 16
ZCH = 2000
NZ = N // ZCH
EB = 128
EP = 819200
ER = EP // EB
RPB = 5
IB = 4
PAD = 2048
FC = 32
BN = 2000
NB = N // BN

_MESH = plsc.VectorSubcoreMesh(
    core_axis_name="c", subcore_axis_name="s", num_cores=NC, num_subcores=NS)
_SC_PARAMS = pltpu.CompilerParams(use_tc_tiling_on_sc=False)


def _zero_acc(zeros_hbm, acc, sid):
    for k in range(2):
        ch = sid + NS * k

        @pl.when(ch < NZ)
        def _():
            pltpu.sync_copy(zeros_hbm, acc.at[pl.ds(ch * ZCH, ZCH)])


def _write_out(acc, out_hbm, sid, slot):
    for k in range(2):
        ch = sid + NS * k

        @pl.when(ch < NZ)
        def _():
            pltpu.sync_copy(acc.at[pl.ds(ch * ZCH, ZCH)],
                            out_hbm.at[pl.ds(slot * N + ch * ZCH, ZCH)])


def _deg_body(dst_hbm, ones_hbm, zeros_hbm, out_hbm, idx_d, ones_v, acc, ssem):
    cid = lax.axis_index("c")
    sid = lax.axis_index("s")
    _zero_acc(zeros_hbm, acc, sid)
    pltpu.sync_copy(ones_hbm, ones_v)
    plsc.subcore_barrier()
    rows_sub = (ER // NC) // NS
    base = cid * (ER // NC) + sid * rows_sub

    def step(b, carry):
        pltpu.sync_copy(dst_hbm.at[pl.ds(base + b * (IB * RPB), IB * RPB)],
                        idx_d)
        descs = [pltpu.async_copy(ones_v, acc.at[idx_d.at[j]], ssem, add=True)
                 for j in range(IB * RPB)]
        for d in descs:
            d.wait()
        return carry

    lax.fori_loop(0, rows_sub // (IB * RPB), step, 0)
    plsc.subcore_barrier()
    _write_out(acc, out_hbm, sid, cid)


_deg_call = pl.kernel(
    _deg_body,
    out_type=jax.ShapeDtypeStruct((NC * N, 8), jnp.float32),
    mesh=_MESH,
    compiler_params=_SC_PARAMS,
    scratch_types=[
        pltpu.VMEM((IB * RPB, EB), jnp.int32),
        pltpu.VMEM((EB, 8), jnp.float32),
        pltpu.VMEM_SHARED((N + PAD, 8), jnp.float32),
        pltpu.SemaphoreType.DMA,
    ],
)


def _make_agg(n_chunks):
    split_edges = n_chunks == 1
    tasks = max(n_chunks // NC, 1)
    rows_core = ER // NC if split_edges else ER
    rows_sub = rows_core // NS
    out_slots = max(n_chunks, 2)

    def body(srcoff_hbm, dst_hbm, table_hbm, zeros_hbm, out_hbm,
             idx_s, idx_d, rows_v, acc, gsem, ssem):
        cid = lax.axis_index("c")
        sid = lax.axis_index("s")
        for k in range(tasks):
            if split_edges:
                chunk = 0
                ebase = cid * rows_core
                slot = cid
            else:
                chunk = cid * tasks + k
                ebase = 0
                slot = chunk
            _zero_acc(zeros_hbm, acc, sid)
            plsc.subcore_barrier()
            base = chunk * ER + ebase + sid * rows_sub
            dbase = ebase + sid * rows_sub

            def step(b, carry):
                @pl.when(b > 0)
                def _():
                    for j in range(RPB):
                        pltpu.make_async_copy(
                            table_hbm.at[pl.ds(0, EB)],
                            rows_v.at[pl.ds(j * EB, EB)], ssem).wait()

                pltpu.sync_copy(
                    srcoff_hbm.at[pl.ds(base + b * (IB * RPB), IB * RPB)], idx_s)
                pltpu.sync_copy(
                    dst_hbm.at[pl.ds(dbase + b * (IB * RPB), IB * RPB)], idx_d)
                for b2 in range(IB):
                    gd = []
                    for j in range(RPB):
                        if b2 > 0:
                            pltpu.make_async_copy(
                                table_hbm.at[pl.ds(0, EB)],
                                rows_v.at[pl.ds(j * EB, EB)], ssem).wait()
                        gd.append(pltpu.async_copy(
                            table_hbm.at[idx_s.at[b2 * RPB + j]],
                            rows_v.at[pl.ds(j * EB, EB)], gsem))
                    for j in range(RPB):
                        gd[j].wait()
                        pltpu.async_copy(rows_v.at[pl.ds(j * EB, EB)],
                                         acc.at[idx_d.at[b2 * RPB + j]],
                                         ssem, add=True)
                return carry

            lax.fori_loop(0, rows_sub // (IB * RPB), step, 0)
            for j in range(RPB):
                pltpu.make_async_copy(table_hbm.at[pl.ds(0, EB)],
                                      rows_v.at[pl.ds(j * EB, EB)], ssem).wait()
            plsc.subcore_barrier()
            _write_out(acc, out_hbm, sid, slot)
            plsc.subcore_barrier()

    return pl.kernel(
        body,
        out_type=jax.ShapeDtypeStruct((out_slots * N, FC), jnp.float32),
        mesh=_MESH,
        compiler_params=_SC_PARAMS,
        scratch_types=[
            pltpu.VMEM((IB * RPB, EB), jnp.int32),
            pltpu.VMEM((IB * RPB, EB), jnp.int32),
            pltpu.VMEM((RPB * EB, FC), jnp.float32),
            pltpu.VMEM_SHARED((N + PAD, FC), jnp.float32),
            pltpu.SemaphoreType.DMA,
            pltpu.SemaphoreType.DMA,
        ],
    )


_agg1 = _make_agg(1)
_agg2 = _make_agg(2)
_agg3 = _make_agg(4)


def _prep_body(degp_ref, x_ref, w1_ref, dinv_ref, hs1_ref):
    deg = degp_ref[0, :, 0:1] + degp_ref[1, :, 0:1] + 1.0
    dinv = lax.rsqrt(deg)
    h = jnp.dot(x_ref[...], w1_ref[...], preferred_element_type=jnp.float32)
    dinv_ref[...] = dinv
    hs1_ref[0] = dinv * h


_prep_call = pl.pallas_call(
    _prep_body,
    grid=(NB,),
    in_specs=[
        pl.BlockSpec((NC, BN, 8), lambda i: (0, i, 0)),
        pl.BlockSpec((BN, 3), lambda i: (i, 0)),
        pl.BlockSpec((3, FC), lambda i: (0, 0)),
    ],
    out_specs=[
        pl.BlockSpec((BN, 1), lambda i: (i, 0)),
        pl.BlockSpec((1, BN, FC), lambda i: (0, i, 0)),
    ],
    out_shape=[
        jax.ShapeDtypeStruct((N, 1), jnp.float32),
        jax.ShapeDtypeStruct((1, N, FC), jnp.float32),
    ],
)


def _layer_t(agg_ref, hs_ref, dinv_ref, b_ref, c, partial):
    a = (agg_ref[0] + agg_ref[1]) if partial else agg_ref[c]
    return dinv_ref[...] * (a + hs_ref[c]) + b_ref[c:c + 1, :]


def _norm_relu(t, s1_acc, s2_acc, gw_ref, gb_ref, ga_ref, c):
    mu = s1_acc[c:c + 1, :] * (1.0 / N)
    m2 = s2_acc[c:c + 1, :] * (1.0 / N)
    al = ga_ref[c:c + 1, :]
    var = m2 - (2.0 * al - al * al) * mu * mu
    u = t - al * mu
    y = gw_ref[c:c + 1, :] * u * lax.rsqrt(var + 1e-5) + gb_ref[c:c + 1, :]
    return jnp.maximum(y, 0.0)


def _stats_phase(agg_ref, hs_ref, dinv_ref, b_ref, s1_acc, s2_acc, C, partial, i):
    @pl.when(i == 0)
    def _():
        s1_acc[...] = jnp.zeros_like(s1_acc)
        s2_acc[...] = jnp.zeros_like(s2_acc)

    for c in range(C):
        t = _layer_t(agg_ref, hs_ref, dinv_ref, b_ref, c, partial)
        s1_acc[c:c + 1, :] += jnp.sum(t, axis=0, keepdims=True)
        s2_acc[c:c + 1, :] += jnp.sum(t * t, axis=0, keepdims=True)


def _make_layer(S, C, partial, C_out):
    def body(agg_ref, hs_ref, dinv_ref, b_ref, gw_ref, gb_ref, ga_ref,
             wn_ref, hsn_ref, s1_acc, s2_acc):
        p = pl.program_id(0)
        i = pl.program_id(1)

        @pl.when(p == 0)
        def _():
            _stats_phase(agg_ref, hs_ref, dinv_ref, b_ref, s1_acc, s2_acc,
                         C, partial, i)

        @pl.when(p == 1)
        def _():
            hn = jnp.zeros((BN, FC * C_out), jnp.float32)
            for c in range(C):
                t = _layer_t(agg_ref, hs_ref, dinv_ref, b_ref, c, partial)
                y = _norm_relu(t, s1_acc, s2_acc, gw_ref, gb_ref, ga_ref, c)
                hn = hn + jnp.dot(y, wn_ref[c * FC:(c + 1) * FC, :],
                                  preferred_element_type=jnp.float32)
            hsn = dinv_ref[...] * hn
            for k in range(C_out):
                hsn_ref[k] = hsn[:, k * FC:(k + 1) * FC]

    return pl.pallas_call(
        body,
        grid=(2, NB),
        in_specs=[
            pl.BlockSpec((S, BN, FC), lambda p, i: (0, i, 0)),
            pl.BlockSpec((C, BN, FC), lambda p, i: (0, i, 0)),
            pl.BlockSpec((BN, 1), lambda p, i: (i, 0)),
            pl.BlockSpec((C, FC), lambda p, i: (0, 0)),
            pl.BlockSpec((C, FC), lambda p, i: (0, 0)),
            pl.BlockSpec((C, FC), lambda p, i: (0, 0)),
            pl.BlockSpec((C, FC), lambda p, i: (0, 0)),
            pl.BlockSpec((C * FC, C_out * FC), lambda p, i: (0, 0)),
        ],
        out_specs=pl.BlockSpec((C_out, BN, FC), lambda p, i: (0, i, 0)),
        out_shape=jax.ShapeDtypeStruct((C_out, N, FC), jnp.float32),
        scratch_shapes=[
            pltpu.VMEM((C, FC), jnp.float32),
            pltpu.VMEM((C, FC), jnp.float32),
        ],
    )


def _pool_body(agg_ref, hs_ref, dinv_ref, b_ref, gw_ref, gb_ref, ga_ref,
               batch_ref, linw_ref, linb_ref, out_ref,
               s1_acc, s2_acc, pool_acc, cnt_acc):
    p = pl.program_id(0)
    i = pl.program_id(1)

    @pl.when(p == 0)
    def _():
        _stats_phase(agg_ref, hs_ref, dinv_ref, b_ref, s1_acc, s2_acc,
                     4, False, i)

    @pl.when(p == 1)
    def _():
        @pl.when(i == 0)
        def _():
            pool_acc[...] = jnp.zeros_like(pool_acc)
            cnt_acc[...] = jnp.zeros_like(cnt_acc)

        gid = lax.broadcasted_iota(jnp.int32, (NG, 1), 0)
        mask = (gid == batch_ref[0]).astype(jnp.float32)
        cnt_acc[...] += jnp.sum(mask, axis=1, keepdims=True)
        for c in range(4):
            t = _layer_t(agg_ref, hs_ref, dinv_ref, b_ref, c, False)
            y = _norm_relu(t, s1_acc, s2_acc, gw_ref, gb_ref, ga_ref, c)
            pool_acc[:, c * FC:(c + 1) * FC] += jnp.dot(
                mask, y, preferred_element_type=jnp.float32)

        @pl.when(i == NB - 1)
        def _():
            pooled = pool_acc[...] / jnp.maximum(cnt_acc[...], 1.0)
            out_ref[...] = jnp.dot(pooled, linw_ref[...],
                                   preferred_element_type=jnp.float32) + linb_ref[...]


_pool_call = pl.pallas_call(
    _pool_body,
    grid=(2, NB),
    in_specs=[
        pl.BlockSpec((4, BN, FC), lambda p, i: (0, i, 0)),
        pl.BlockSpec((4, BN, FC), lambda p, i: (0, i, 0)),
        pl.BlockSpec((BN, 1), lambda p, i: (i, 0)),
        pl.BlockSpec((4, FC), lambda p, i: (0, 0)),
        pl.BlockSpec((4, FC), lambda p, i: (0, 0)),
        pl.BlockSpec((4, FC), lambda p, i: (0, 0)),
        pl.BlockSpec((4, FC), lambda p, i: (0, 0)),
        pl.BlockSpec((1, 1, BN), lambda p, i: (i, 0, 0)),
        pl.BlockSpec((4 * FC, 3), lambda p, i: (0, 0)),
        pl.BlockSpec((1, 3), lambda p, i: (0, 0)),
    ],
    out_specs=pl.BlockSpec((NG, 3), lambda p, i: (0, 0)),
    out_shape=jax.ShapeDtypeStruct((NG, 3), jnp.float32),
    scratch_shapes=[
        pltpu.VMEM((4, FC), jnp.float32),
        pltpu.VMEM((4, FC), jnp.float32),
        pltpu.VMEM((NG, 4 * FC), jnp.float32),
        pltpu.VMEM((NG, 1), jnp.float32),
    ],
)

_layer1 = _make_layer(2, 1, True, 2)
_layer2 = _make_layer(2, 2, False, 4)


def kernel(x, edge_index, batch, W1, b1, g1w, g1b, g1a, W2, b2, g2w, g2b, g2a,
           W3, b3, g3w, g3b, g3a, linW, linb):
    pad = EP - E
    src = jnp.concatenate(
        [edge_index[0], jnp.arange(pad, dtype=jnp.int32) % 1024])
    dst2d = jnp.concatenate(
        [edge_index[1],
         N + (jnp.arange(pad, dtype=jnp.int32) % PAD)]).reshape(ER, EB)
    srcoff = (src[None, :]
              + (jnp.arange(4, dtype=jnp.int32) * N)[:, None]).reshape(4 * ER, EB)
    zeros_fc = jnp.zeros((ZCH, FC), jnp.float32)
    zeros_8 = jnp.zeros((ZCH, 8), jnp.float32)
    ones_8 = jnp.ones((EB, 8), jnp.float32)

    degp = _deg_call(dst2d, ones_8, zeros_8).reshape(NC, N, 8)
    dinv, hs1 = _prep_call(degp, x, W1)

    agg1 = _agg1(srcoff, dst2d, hs1.reshape(N, FC), zeros_fc).reshape(2, N, FC)
    hs2 = _layer1(agg1, hs1, dinv, b1.reshape(1, FC),
                  g1w.reshape(1, FC), g1b.reshape(1, FC), g1a.reshape(1, FC), W2)

    agg2 = _agg2(srcoff, dst2d, hs2.reshape(2 * N, FC),
                 zeros_fc).reshape(2, N, FC)
    hs3 = _layer2(agg2, hs2, dinv, b2.reshape(2, FC),
                  g2w.reshape(2, FC), g2b.reshape(2, FC), g2a.reshape(2, FC), W3)

    agg3 = _agg3(srcoff, dst2d, hs3.reshape(4 * N, FC),
                 zeros_fc).reshape(4, N, FC)
    out = _pool_call(agg3, hs3, dinv, b3.reshape(4, FC),
                     g3w.reshape(4, FC), g3b.reshape(4, FC), g3a.reshape(4, FC),
                     batch.reshape(NB, 1, BN), linW, linb.reshape(1, 3))
    return out


# speedup vs baseline: 1.1502x; 1.0614x over previous
import functools

import jax
import jax.numpy as jnp
from jax import lax
from jax.experimental import pallas as pl
from jax.experimental.pallas import tpu as pltpu
from jax.experimental.pallas import tpu_sc as plsc

N = 50000
E = 800000
NG = 128
NC = 2
NS = 16
ZCH = 2000
NZ = N // ZCH
EB = 128
EP = 819200
ER = EP // EB
RPB = 5
IB = 4
PAD = 2048
FC = 32
BN = 2000
NB = N // BN

_MESH = plsc.VectorSubcoreMesh(
    core_axis_name="c", subcore_axis_name="s", num_cores=NC, num_subcores=NS)
_SC_PARAMS = pltpu.CompilerParams(use_tc_tiling_on_sc=False)


def _zero_acc(zeros_hbm, acc, sid):
    for k in range(2):
        ch = sid + NS * k

        @pl.when(ch < NZ)
        def _():
            pltpu.sync_copy(zeros_hbm, acc.at[pl.ds(ch * ZCH, ZCH)])


def _write_out(acc, out_hbm, sid, slot):
    for k in range(2):
        ch = sid + NS * k

        @pl.when(ch < NZ)
        def _():
            pltpu.sync_copy(acc.at[pl.ds(ch * ZCH, ZCH)],
                            out_hbm.at[pl.ds(slot * N + ch * ZCH, ZCH)])


def _deg_body(dst_hbm, ones_hbm, zeros_hbm, out_hbm, idx_d, ones_v, acc, ssem):
    cid = lax.axis_index("c")
    sid = lax.axis_index("s")
    _zero_acc(zeros_hbm, acc, sid)
    pltpu.sync_copy(ones_hbm, ones_v)
    plsc.subcore_barrier()
    rows_sub = (ER // NC) // NS
    base = cid * (ER // NC) + sid * rows_sub

    def step(b, carry):
        pltpu.sync_copy(dst_hbm.at[pl.ds(base + b * (IB * RPB), IB * RPB)],
                        idx_d)
        descs = [pltpu.async_copy(ones_v, acc.at[idx_d.at[j]], ssem, add=True)
                 for j in range(IB * RPB)]
        for d in descs:
            d.wait()
        return carry

    lax.fori_loop(0, rows_sub // (IB * RPB), step, 0)
    plsc.subcore_barrier()
    _write_out(acc, out_hbm, sid, cid)


_deg_call = pl.kernel(
    _deg_body,
    out_type=jax.ShapeDtypeStruct((NC * N, 8), jnp.float32),
    mesh=_MESH,
    compiler_params=_SC_PARAMS,
    scratch_types=[
        pltpu.VMEM((IB * RPB, EB), jnp.int32),
        pltpu.VMEM((EB, 8), jnp.float32),
        pltpu.VMEM_SHARED((N + PAD, 8), jnp.float32),
        pltpu.SemaphoreType.DMA,
    ],
)


def _make_agg(n_chunks):
    split_edges = n_chunks == 1
    tasks = max(n_chunks // NC, 1)
    rows_core = ER // NC if split_edges else ER
    rows_sub = rows_core // NS
    out_slots = max(n_chunks, 2)

    def body(srcoff_hbm, dst_hbm, table_hbm, zeros_hbm, out_hbm,
             idx_s, idx_d, rows_v, acc, gsem, ssem):
        cid = lax.axis_index("c")
        sid = lax.axis_index("s")
        for k in range(tasks):
            if split_edges:
                chunk = 0
                ebase = cid * rows_core
                slot = cid
            else:
                chunk = cid * tasks + k
                ebase = 0
                slot = chunk
            if split_edges:
                @pl.when(cid == 0)
                def _():
                    for kk in range(2):
                        ch = sid + NS * kk

                        @pl.when(ch < NZ)
                        def _(ch=ch):
                            pltpu.sync_copy(
                                table_hbm.at[pl.ds(ch * ZCH, ZCH)],
                                acc.at[pl.ds(ch * ZCH, ZCH)])

                @pl.when(cid == 1)
                def _():
                    _zero_acc(zeros_hbm, acc, sid)
            else:
                for kk in range(2):
                    ch = sid + NS * kk

                    @pl.when(ch < NZ)
                    def _(ch=ch):
                        pltpu.sync_copy(
                            table_hbm.at[pl.ds(chunk * N + ch * ZCH, ZCH)],
                            acc.at[pl.ds(ch * ZCH, ZCH)])
            plsc.subcore_barrier()
            base = chunk * ER + ebase + sid * rows_sub
            dbase = ebase + sid * rows_sub

            def step(b, carry):
                @pl.when(b > 0)
                def _():
                    for j in range(RPB):
                        pltpu.make_async_copy(
                            table_hbm.at[pl.ds(0, EB)],
                            rows_v.at[pl.ds(j * EB, EB)], ssem).wait()

                pltpu.sync_copy(
                    srcoff_hbm.at[pl.ds(base + b * (IB * RPB), IB * RPB)], idx_s)
                pltpu.sync_copy(
                    dst_hbm.at[pl.ds(dbase + b * (IB * RPB), IB * RPB)], idx_d)
                for b2 in range(IB):
                    gd = []
                    for j in range(RPB):
                        if b2 > 0:
                            pltpu.make_async_copy(
                                table_hbm.at[pl.ds(0, EB)],
                                rows_v.at[pl.ds(j * EB, EB)], ssem).wait()
                        gd.append(pltpu.async_copy(
                            table_hbm.at[idx_s.at[b2 * RPB + j]],
                            rows_v.at[pl.ds(j * EB, EB)], gsem))
                    for j in range(RPB):
                        gd[j].wait()
                        pltpu.async_copy(rows_v.at[pl.ds(j * EB, EB)],
                                         acc.at[idx_d.at[b2 * RPB + j]],
                                         ssem, add=True)
                return carry

            lax.fori_loop(0, rows_sub // (IB * RPB), step, 0)
            for j in range(RPB):
                pltpu.make_async_copy(table_hbm.at[pl.ds(0, EB)],
                                      rows_v.at[pl.ds(j * EB, EB)], ssem).wait()
            plsc.subcore_barrier()
            _write_out(acc, out_hbm, sid, slot)
            plsc.subcore_barrier()

    return pl.kernel(
        body,
        out_type=jax.ShapeDtypeStruct((out_slots * N, FC), jnp.float32),
        mesh=_MESH,
        compiler_params=_SC_PARAMS,
        scratch_types=[
            pltpu.VMEM((IB * RPB, EB), jnp.int32),
            pltpu.VMEM((IB * RPB, EB), jnp.int32),
            pltpu.VMEM((RPB * EB, FC), jnp.float32),
            pltpu.VMEM_SHARED((N + PAD, FC), jnp.float32),
            pltpu.SemaphoreType.DMA,
            pltpu.SemaphoreType.DMA,
        ],
    )


_agg1 = _make_agg(1)
_agg2 = _make_agg(2)
_agg3 = _make_agg(4)


def _prep_body(degp_ref, x_ref, w1_ref, dinv_ref, hs1_ref):
    deg = degp_ref[0, :, 0:1] + degp_ref[1, :, 0:1] + 1.0
    dinv = lax.rsqrt(deg)
    h = jnp.dot(x_ref[...], w1_ref[...], preferred_element_type=jnp.float32)
    dinv_ref[...] = dinv
    hs1_ref[0] = dinv * h


_prep_call = pl.pallas_call(
    _prep_body,
    grid=(NB,),
    in_specs=[
        pl.BlockSpec((NC, BN, 8), lambda i: (0, i, 0)),
        pl.BlockSpec((BN, 3), lambda i: (i, 0)),
        pl.BlockSpec((3, FC), lambda i: (0, 0)),
    ],
    out_specs=[
        pl.BlockSpec((BN, 1), lambda i: (i, 0)),
        pl.BlockSpec((1, BN, FC), lambda i: (0, i, 0)),
    ],
    out_shape=[
        jax.ShapeDtypeStruct((N, 1), jnp.float32),
        jax.ShapeDtypeStruct((1, N, FC), jnp.float32),
    ],
)


def _layer_t(agg_ref, dinv_ref, b_ref, c, partial):
    a = (agg_ref[0] + agg_ref[1]) if partial else agg_ref[c]
    return dinv_ref[...] * a + b_ref[c:c + 1, :]


def _norm_relu(t, s1_acc, s2_acc, gw_ref, gb_ref, ga_ref, c):
    mu = s1_acc[c:c + 1, :] * (1.0 / N)
    m2 = s2_acc[c:c + 1, :] * (1.0 / N)
    al = ga_ref[c:c + 1, :]
    var = m2 - (2.0 * al - al * al) * mu * mu
    u = t - al * mu
    y = gw_ref[c:c + 1, :] * u * lax.rsqrt(var + 1e-5) + gb_ref[c:c + 1, :]
    return jnp.maximum(y, 0.0)


def _stats_phase(agg_ref, dinv_ref, b_ref, s1_acc, s2_acc, C, partial, i):
    @pl.when(i == 0)
    def _():
        s1_acc[...] = jnp.zeros_like(s1_acc)
        s2_acc[...] = jnp.zeros_like(s2_acc)

    for c in range(C):
        t = _layer_t(agg_ref, dinv_ref, b_ref, c, partial)
        s1_acc[c:c + 1, :] += jnp.sum(t, axis=0, keepdims=True)
        s2_acc[c:c + 1, :] += jnp.sum(t * t, axis=0, keepdims=True)


def _make_layer(S, C, partial, C_out):
    def body(agg_ref, dinv_ref, b_ref, gw_ref, gb_ref, ga_ref,
             wn_ref, hsn_ref, s1_acc, s2_acc):
        p = pl.program_id(0)
        i = pl.program_id(1)

        @pl.when(p == 0)
        def _():
            _stats_phase(agg_ref, dinv_ref, b_ref, s1_acc, s2_acc,
                         C, partial, i)

        @pl.when(p == 1)
        def _():
            hn = jnp.zeros((BN, FC * C_out), jnp.float32)
            for c in range(C):
                t = _layer_t(agg_ref, dinv_ref, b_ref, c, partial)
                y = _norm_relu(t, s1_acc, s2_acc, gw_ref, gb_ref, ga_ref, c)
                hn = hn + jnp.dot(y, wn_ref[c * FC:(c + 1) * FC, :],
                                  preferred_element_type=jnp.float32)
            hsn = dinv_ref[...] * hn
            for k in range(C_out):
                hsn_ref[k] = hsn[:, k * FC:(k + 1) * FC]

    return pl.pallas_call(
        body,
        grid=(2, NB),
        in_specs=[
            pl.BlockSpec((S, BN, FC), lambda p, i: (0, i, 0)),
            pl.BlockSpec((BN, 1), lambda p, i: (i, 0)),
            pl.BlockSpec((C, FC), lambda p, i: (0, 0)),
            pl.BlockSpec((C, FC), lambda p, i: (0, 0)),
            pl.BlockSpec((C, FC), lambda p, i: (0, 0)),
            pl.BlockSpec((C, FC), lambda p, i: (0, 0)),
            pl.BlockSpec((C * FC, C_out * FC), lambda p, i: (0, 0)),
        ],
        out_specs=pl.BlockSpec((C_out, BN, FC), lambda p, i: (0, i, 0)),
        out_shape=jax.ShapeDtypeStruct((C_out, N, FC), jnp.float32),
        scratch_shapes=[
            pltpu.VMEM((C, FC), jnp.float32),
            pltpu.VMEM((C, FC), jnp.float32),
        ],
    )


def _pool_body(agg_ref, dinv_ref, b_ref, gw_ref, gb_ref, ga_ref,
               batch_ref, linw_ref, linb_ref, out_ref,
               s1_acc, s2_acc, pool_acc, cnt_acc):
    p = pl.program_id(0)
    i = pl.program_id(1)

    @pl.when(p == 0)
    def _():
        _stats_phase(agg_ref, dinv_ref, b_ref, s1_acc, s2_acc,
                     4, False, i)

    @pl.when(p == 1)
    def _():
        @pl.when(i == 0)
        def _():
            pool_acc[...] = jnp.zeros_like(pool_acc)
            cnt_acc[...] = jnp.zeros_like(cnt_acc)

        gid = lax.broadcasted_iota(jnp.int32, (NG, 1), 0)
        mask = (gid == batch_ref[0]).astype(jnp.float32)
        cnt_acc[...] += jnp.sum(mask, axis=1, keepdims=True)
        for c in range(4):
            t = _layer_t(agg_ref, dinv_ref, b_ref, c, False)
            y = _norm_relu(t, s1_acc, s2_acc, gw_ref, gb_ref, ga_ref, c)
            pool_acc[:, c * FC:(c + 1) * FC] += jnp.dot(
                mask, y, preferred_element_type=jnp.float32)

        @pl.when(i == NB - 1)
        def _():
            pooled = pool_acc[...] / jnp.maximum(cnt_acc[...], 1.0)
            out_ref[...] = jnp.dot(pooled, linw_ref[...],
                                   preferred_element_type=jnp.float32) + linb_ref[...]


_pool_call = pl.pallas_call(
    _pool_body,
    grid=(2, NB),
    in_specs=[
        pl.BlockSpec((4, BN, FC), lambda p, i: (0, i, 0)),
        pl.BlockSpec((BN, 1), lambda p, i: (i, 0)),
        pl.BlockSpec((4, FC), lambda p, i: (0, 0)),
        pl.BlockSpec((4, FC), lambda p, i: (0, 0)),
        pl.BlockSpec((4, FC), lambda p, i: (0, 0)),
        pl.BlockSpec((4, FC), lambda p, i: (0, 0)),
        pl.BlockSpec((1, 1, BN), lambda p, i: (i, 0, 0)),
        pl.BlockSpec((4 * FC, 3), lambda p, i: (0, 0)),
        pl.BlockSpec((1, 3), lambda p, i: (0, 0)),
    ],
    out_specs=pl.BlockSpec((NG, 3), lambda p, i: (0, 0)),
    out_shape=jax.ShapeDtypeStruct((NG, 3), jnp.float32),
    scratch_shapes=[
        pltpu.VMEM((4, FC), jnp.float32),
        pltpu.VMEM((4, FC), jnp.float32),
        pltpu.VMEM((NG, 4 * FC), jnp.float32),
        pltpu.VMEM((NG, 1), jnp.float32),
    ],
)

_layer1 = _make_layer(2, 1, True, 2)
_layer2 = _make_layer(2, 2, False, 4)


def kernel(x, edge_index, batch, W1, b1, g1w, g1b, g1a, W2, b2, g2w, g2b, g2a,
           W3, b3, g3w, g3b, g3a, linW, linb):
    pad = EP - E
    src = jnp.concatenate(
        [edge_index[0], jnp.arange(pad, dtype=jnp.int32) % 1024])
    dst2d = jnp.concatenate(
        [edge_index[1],
         N + (jnp.arange(pad, dtype=jnp.int32) % PAD)]).reshape(ER, EB)
    srcoff = (src[None, :]
              + (jnp.arange(4, dtype=jnp.int32) * N)[:, None]).reshape(4 * ER, EB)
    zeros_fc = jnp.zeros((ZCH, FC), jnp.float32)
    zeros_8 = jnp.zeros((ZCH, 8), jnp.float32)
    ones_8 = jnp.ones((EB, 8), jnp.float32)

    degp = _deg_call(dst2d, ones_8, zeros_8).reshape(NC, N, 8)
    dinv, hs1 = _prep_call(degp, x, W1)

    agg1 = _agg1(srcoff, dst2d, hs1.reshape(N, FC), zeros_fc).reshape(2, N, FC)
    hs2 = _layer1(agg1, dinv, b1.reshape(1, FC),
                  g1w.reshape(1, FC), g1b.reshape(1, FC), g1a.reshape(1, FC), W2)

    agg2 = _agg2(srcoff, dst2d, hs2.reshape(2 * N, FC),
                 zeros_fc).reshape(2, N, FC)
    hs3 = _layer2(agg2, dinv, b2.reshape(2, FC),
                  g2w.reshape(2, FC), g2b.reshape(2, FC), g2a.reshape(2, FC), W3)

    agg3 = _agg3(srcoff, dst2d, hs3.reshape(4 * N, FC),
                 zeros_fc).reshape(4, N, FC)
    out = _pool_call(agg3, dinv, b3.reshape(4, FC),
                     g3w.reshape(4, FC), g3b.reshape(4, FC), g3a.reshape(4, FC),
                     batch.reshape(NB, 1, BN), linW, linb.reshape(1, 3))
    return out


# speedup vs baseline: 1.1788x; 1.0249x over previous
import functools

import jax
import jax.numpy as jnp
from jax import lax
from jax.experimental import pallas as pl
from jax.experimental.pallas import tpu as pltpu
from jax.experimental.pallas import tpu_sc as plsc

N = 50000
E = 800000
NG = 128
NC = 2
NS = 16
ZCH = 2000
NZ = N // ZCH
EB = 128
EP = 819200
ER = EP // EB
RPB = 5
IB = 4
PAD = 2048
FC = 32
BN = 2000
NB = N // BN

_MESH = plsc.VectorSubcoreMesh(
    core_axis_name="c", subcore_axis_name="s", num_cores=NC, num_subcores=NS)
_SC_PARAMS = pltpu.CompilerParams(use_tc_tiling_on_sc=False)


def _zero_acc(zeros_hbm, acc, sid):
    for k in range(2):
        ch = sid + NS * k

        @pl.when(ch < NZ)
        def _():
            pltpu.sync_copy(zeros_hbm, acc.at[pl.ds(ch * ZCH, ZCH)])


def _write_out(acc, out_hbm, sid, slot):
    for k in range(2):
        ch = sid + NS * k

        @pl.when(ch < NZ)
        def _():
            pltpu.sync_copy(acc.at[pl.ds(ch * ZCH, ZCH)],
                            out_hbm.at[pl.ds(slot * N + ch * ZCH, ZCH)])


def _deg_body(dst_hbm, ones_hbm, zeros_hbm, out_hbm, idx_d, ones_v, acc, ssem):
    cid = lax.axis_index("c")
    sid = lax.axis_index("s")
    _zero_acc(zeros_hbm, acc, sid)
    pltpu.sync_copy(ones_hbm, ones_v)
    plsc.subcore_barrier()
    rows_sub = (ER // NC) // NS
    base = cid * (ER // NC) + sid * rows_sub

    def step(b, carry):
        pltpu.sync_copy(dst_hbm.at[pl.ds(base + b * (IB * RPB), IB * RPB)],
                        idx_d)
        descs = [pltpu.async_copy(ones_v, acc.at[idx_d.at[j]], ssem, add=True)
                 for j in range(IB * RPB)]
        for d in descs:
            d.wait()
        return carry

    lax.fori_loop(0, rows_sub // (IB * RPB), step, 0)
    plsc.subcore_barrier()
    _write_out(acc, out_hbm, sid, cid)


_deg_call = pl.kernel(
    _deg_body,
    out_type=jax.ShapeDtypeStruct((NC * N, 8), jnp.float32),
    mesh=_MESH,
    compiler_params=_SC_PARAMS,
    scratch_types=[
        pltpu.VMEM((IB * RPB, EB), jnp.int32),
        pltpu.VMEM((EB, 8), jnp.float32),
        pltpu.VMEM_SHARED((N + PAD, 8), jnp.float32),
        pltpu.SemaphoreType.DMA,
    ],
)


def _make_agg(n_chunks):
    split_edges = n_chunks == 1
    tasks = max(n_chunks // NC, 1)
    rows_core = ER // NC if split_edges else ER
    rows_sub = rows_core // NS
    out_slots = max(n_chunks, 2)

    def body(srcoff_hbm, dst_hbm, table_hbm, zeros_hbm, out_hbm,
             idx_s, idx_d, rows_v, acc, gsem, ssem):
        cid = lax.axis_index("c")
        sid = lax.axis_index("s")
        for k in range(tasks):
            if split_edges:
                chunk = 0
                ebase = cid * rows_core
                slot = cid
            else:
                chunk = cid * tasks + k
                ebase = 0
                slot = chunk
            if split_edges:
                @pl.when(cid == 0)
                def _():
                    for kk in range(2):
                        ch = sid + NS * kk

                        @pl.when(ch < NZ)
                        def _(ch=ch):
                            pltpu.sync_copy(
                                table_hbm.at[pl.ds(ch * ZCH, ZCH)],
                                acc.at[pl.ds(ch * ZCH, ZCH)])

                @pl.when(cid == 1)
                def _():
                    _zero_acc(zeros_hbm, acc, sid)
            else:
                for kk in range(2):
                    ch = sid + NS * kk

                    @pl.when(ch < NZ)
                    def _(ch=ch):
                        pltpu.sync_copy(
                            table_hbm.at[pl.ds(chunk * N + ch * ZCH, ZCH)],
                            acc.at[pl.ds(ch * ZCH, ZCH)])
            plsc.subcore_barrier()
            base = chunk * ER + ebase + sid * rows_sub
            dbase = ebase + sid * rows_sub

            def step(b, carry):
                @pl.when(b > 0)
                def _():
                    for j in range(RPB):
                        pltpu.make_async_copy(
                            table_hbm.at[pl.ds(0, EB)],
                            rows_v.at[pl.ds(j * EB, EB)], ssem).wait()

                pltpu.sync_copy(
                    srcoff_hbm.at[pl.ds(base + b * (IB * RPB), IB * RPB)], idx_s)
                pltpu.sync_copy(
                    dst_hbm.at[pl.ds(dbase + b * (IB * RPB), IB * RPB)], idx_d)
                for b2 in range(IB):
                    gd = []
                    for j in range(RPB):
                        if b2 > 0:
                            pltpu.make_async_copy(
                                table_hbm.at[pl.ds(0, EB)],
                                rows_v.at[pl.ds(j * EB, EB)], ssem).wait()
                        gd.append(pltpu.async_copy(
                            table_hbm.at[idx_s.at[b2 * RPB + j]],
                            rows_v.at[pl.ds(j * EB, EB)], gsem))
                    for j in range(RPB):
                        gd[j].wait()
                        pltpu.async_copy(rows_v.at[pl.ds(j * EB, EB)],
                                         acc.at[idx_d.at[b2 * RPB + j]],
                                         ssem, add=True)
                return carry

            lax.fori_loop(0, rows_sub // (IB * RPB), step, 0)
            for j in range(RPB):
                pltpu.make_async_copy(table_hbm.at[pl.ds(0, EB)],
                                      rows_v.at[pl.ds(j * EB, EB)], ssem).wait()
            plsc.subcore_barrier()
            _write_out(acc, out_hbm, sid, slot)
            plsc.subcore_barrier()

    return pl.kernel(
        body,
        out_type=jax.ShapeDtypeStruct((out_slots * N, FC), jnp.float32),
        mesh=_MESH,
        compiler_params=_SC_PARAMS,
        scratch_types=[
            pltpu.VMEM((IB * RPB, EB), jnp.int32),
            pltpu.VMEM((IB * RPB, EB), jnp.int32),
            pltpu.VMEM((RPB * EB, FC), jnp.float32),
            pltpu.VMEM_SHARED((N + PAD, FC), jnp.float32),
            pltpu.SemaphoreType.DMA,
            pltpu.SemaphoreType.DMA,
        ],
    )


_agg1 = _make_agg(1)
_agg2 = _make_agg(2)
_agg3 = _make_agg(4)


def _prep_body(degp_ref, x_ref, w1_ref, hs1_ref):
    deg = degp_ref[0, :, 0:1] + degp_ref[1, :, 0:1] + 1.0
    dinv = lax.rsqrt(deg)
    h = jnp.dot(x_ref[...], w1_ref[...], preferred_element_type=jnp.float32)
    hs1_ref[0] = dinv * h


_prep_call = pl.pallas_call(
    _prep_body,
    grid=(NB,),
    in_specs=[
        pl.BlockSpec((NC, BN, 8), lambda i: (0, i, 0)),
        pl.BlockSpec((BN, 3), lambda i: (i, 0)),
        pl.BlockSpec((3, FC), lambda i: (0, 0)),
    ],
    out_specs=pl.BlockSpec((1, BN, FC), lambda i: (0, i, 0)),
    out_shape=jax.ShapeDtypeStruct((1, N, FC), jnp.float32),
)


def _layer_t(agg_ref, dinv_ref, b_ref, c, partial):
    a = (agg_ref[0] + agg_ref[1]) if partial else agg_ref[c]
    dinv = dinv_ref[...].reshape(BN, 1)
    return dinv * a + b_ref[c:c + 1, :]


def _norm_relu(t, s1_acc, s2_acc, gw_ref, gb_ref, ga_ref, c):
    mu = s1_acc[c:c + 1, :] * (1.0 / N)
    m2 = s2_acc[c:c + 1, :] * (1.0 / N)
    al = ga_ref[c:c + 1, :]
    var = m2 - (2.0 * al - al * al) * mu * mu
    u = t - al * mu
    y = gw_ref[c:c + 1, :] * u * lax.rsqrt(var + 1e-5) + gb_ref[c:c + 1, :]
    return jnp.maximum(y, 0.0)


def _stats_phase(agg_ref, dinv_ref, b_ref, s1_acc, s2_acc, C, partial, i):
    @pl.when(i == 0)
    def _():
        s1_acc[...] = jnp.zeros_like(s1_acc)
        s2_acc[...] = jnp.zeros_like(s2_acc)

    for c in range(C):
        t = _layer_t(agg_ref, dinv_ref, b_ref, c, partial)
        s1_acc[c:c + 1, :] += jnp.sum(t, axis=0, keepdims=True)
        s2_acc[c:c + 1, :] += jnp.sum(t * t, axis=0, keepdims=True)


def _make_layer(S, C, partial, C_out):
    def body(agg_ref, dinv_ref, b_ref, gw_ref, gb_ref, ga_ref,
             wn_ref, hsn_ref, s1_acc, s2_acc):
        p = pl.program_id(0)
        i = pl.program_id(1)

        @pl.when(p == 0)
        def _():
            _stats_phase(agg_ref, dinv_ref, b_ref, s1_acc, s2_acc,
                         C, partial, i)

        @pl.when(p == 1)
        def _():
            hn = jnp.zeros((BN, FC * C_out), jnp.float32)
            for c in range(C):
                t = _layer_t(agg_ref, dinv_ref, b_ref, c, partial)
                y = _norm_relu(t, s1_acc, s2_acc, gw_ref, gb_ref, ga_ref, c)
                hn = hn + jnp.dot(y, wn_ref[c * FC:(c + 1) * FC, :],
                                  preferred_element_type=jnp.float32)
            hsn = dinv_ref[...].reshape(BN, 1) * hn
            for k in range(C_out):
                hsn_ref[k] = hsn[:, k * FC:(k + 1) * FC]

    return pl.pallas_call(
        body,
        grid=(2, NB),
        in_specs=[
            pl.BlockSpec((S, BN, FC), lambda p, i: (0, i, 0)),
            pl.BlockSpec((1, 1, BN), lambda p, i: (i, 0, 0)),
            pl.BlockSpec((C, FC), lambda p, i: (0, 0)),
            pl.BlockSpec((C, FC), lambda p, i: (0, 0)),
            pl.BlockSpec((C, FC), lambda p, i: (0, 0)),
            pl.BlockSpec((C, FC), lambda p, i: (0, 0)),
            pl.BlockSpec((C * FC, C_out * FC), lambda p, i: (0, 0)),
        ],
        out_specs=pl.BlockSpec((C_out, BN, FC), lambda p, i: (0, i, 0)),
        out_shape=jax.ShapeDtypeStruct((C_out, N, FC), jnp.float32),
        scratch_shapes=[
            pltpu.VMEM((C, FC), jnp.float32),
            pltpu.VMEM((C, FC), jnp.float32),
        ],
    )


def _pool_body(agg_ref, dinv_ref, b_ref, gw_ref, gb_ref, ga_ref,
               batch_ref, linw_ref, linb_ref, out_ref,
               s1_acc, s2_acc, pool_acc, cnt_acc):
    p = pl.program_id(0)
    i = pl.program_id(1)

    @pl.when(p == 0)
    def _():
        _stats_phase(agg_ref, dinv_ref, b_ref, s1_acc, s2_acc,
                     4, False, i)

    @pl.when(p == 1)
    def _():
        @pl.when(i == 0)
        def _():
            pool_acc[...] = jnp.zeros_like(pool_acc)
            cnt_acc[...] = jnp.zeros_like(cnt_acc)

        gid = lax.broadcasted_iota(jnp.int32, (NG, 1), 0)
        mask = (gid == batch_ref[0]).astype(jnp.float32)
        cnt_acc[...] += jnp.sum(mask, axis=1, keepdims=True)
        for c in range(4):
            t = _layer_t(agg_ref, dinv_ref, b_ref, c, False)
            y = _norm_relu(t, s1_acc, s2_acc, gw_ref, gb_ref, ga_ref, c)
            pool_acc[:, c * FC:(c + 1) * FC] += jnp.dot(
                mask, y, preferred_element_type=jnp.float32)

        @pl.when(i == NB - 1)
        def _():
            pooled = pool_acc[...] / jnp.maximum(cnt_acc[...], 1.0)
            out_ref[...] = jnp.dot(pooled, linw_ref[...],
                                   preferred_element_type=jnp.float32) + linb_ref[...]


_pool_call = pl.pallas_call(
    _pool_body,
    grid=(2, NB),
    in_specs=[
        pl.BlockSpec((4, BN, FC), lambda p, i: (0, i, 0)),
        pl.BlockSpec((1, 1, BN), lambda p, i: (i, 0, 0)),
        pl.BlockSpec((4, FC), lambda p, i: (0, 0)),
        pl.BlockSpec((4, FC), lambda p, i: (0, 0)),
        pl.BlockSpec((4, FC), lambda p, i: (0, 0)),
        pl.BlockSpec((4, FC), lambda p, i: (0, 0)),
        pl.BlockSpec((1, 1, BN), lambda p, i: (i, 0, 0)),
        pl.BlockSpec((4 * FC, 3), lambda p, i: (0, 0)),
        pl.BlockSpec((1, 3), lambda p, i: (0, 0)),
    ],
    out_specs=pl.BlockSpec((NG, 3), lambda p, i: (0, 0)),
    out_shape=jax.ShapeDtypeStruct((NG, 3), jnp.float32),
    scratch_shapes=[
        pltpu.VMEM((4, FC), jnp.float32),
        pltpu.VMEM((4, FC), jnp.float32),
        pltpu.VMEM((NG, 4 * FC), jnp.float32),
        pltpu.VMEM((NG, 1), jnp.float32),
    ],
)

_layer1 = _make_layer(2, 1, True, 2)
_layer2 = _make_layer(2, 2, False, 4)


def kernel(x, edge_index, batch, W1, b1, g1w, g1b, g1a, W2, b2, g2w, g2b, g2a,
           W3, b3, g3w, g3b, g3a, linW, linb):
    pad = EP - E
    src = jnp.concatenate(
        [edge_index[0], jnp.arange(pad, dtype=jnp.int32) % 1024])
    dst2d = jnp.concatenate(
        [edge_index[1],
         N + (jnp.arange(pad, dtype=jnp.int32) % PAD)]).reshape(ER, EB)
    srcoff = (src[None, :]
              + (jnp.arange(4, dtype=jnp.int32) * N)[:, None]).reshape(4 * ER, EB)
    zeros_fc = jnp.zeros((ZCH, FC), jnp.float32)
    zeros_8 = jnp.zeros((ZCH, 8), jnp.float32)
    ones_8 = jnp.ones((EB, 8), jnp.float32)

    degp = _deg_call(dst2d, ones_8, zeros_8).reshape(NC, N, 8)
    dinv = lax.rsqrt(degp[0, :, 0] + degp[1, :, 0] + 1.0).reshape(NB, 1, BN)
    hs1 = _prep_call(degp, x, W1)

    agg1 = _agg1(srcoff, dst2d, hs1.reshape(N, FC), zeros_fc).reshape(2, N, FC)
    hs2 = _layer1(agg1, dinv, b1.reshape(1, FC),
                  g1w.reshape(1, FC), g1b.reshape(1, FC), g1a.reshape(1, FC), W2)

    agg2 = _agg2(srcoff, dst2d, hs2.reshape(2 * N, FC),
                 zeros_fc).reshape(2, N, FC)
    hs3 = _layer2(agg2, dinv, b2.reshape(2, FC),
                  g2w.reshape(2, FC), g2b.reshape(2, FC), g2a.reshape(2, FC), W3)

    agg3 = _agg3(srcoff, dst2d, hs3.reshape(4 * N, FC),
                 zeros_fc).reshape(4, N, FC)
    out = _pool_call(agg3, dinv, b3.reshape(4, FC),
                     g3w.reshape(4, FC), g3b.reshape(4, FC), g3a.reshape(4, FC),
                     batch.reshape(NB, 1, BN), linW, linb.reshape(1, 3))
    return out


# speedup vs baseline: 1.1943x; 1.0131x over previous
import functools

import jax
import jax.numpy as jnp
from jax import lax
from jax.experimental import pallas as pl
from jax.experimental.pallas import tpu as pltpu
from jax.experimental.pallas import tpu_sc as plsc

N = 50000
E = 800000
NG = 128
NC = 2
NS = 16
ZCH = 2000
NZ = N // ZCH
EB = 128
EP = 819200
ER = EP // EB
RPB = 5
IB = 5
PAD = 1024
FC = 32
BN = 2000
NB = N // BN

_MESH = plsc.VectorSubcoreMesh(
    core_axis_name="c", subcore_axis_name="s", num_cores=NC, num_subcores=NS)
_SC_PARAMS = pltpu.CompilerParams(use_tc_tiling_on_sc=False)


def _zero_acc(zeros_hbm, acc, sid):
    for k in range(2):
        ch = sid + NS * k

        @pl.when(ch < NZ)
        def _():
            pltpu.sync_copy(zeros_hbm, acc.at[pl.ds(ch * ZCH, ZCH)])


def _write_out(acc, out_hbm, sid, slot):
    for k in range(2):
        ch = sid + NS * k

        @pl.when(ch < NZ)
        def _():
            pltpu.sync_copy(acc.at[pl.ds(ch * ZCH, ZCH)],
                            out_hbm.at[pl.ds(slot * N + ch * ZCH, ZCH)])


def _deg_body(dst_hbm, ones_hbm, zeros_hbm, out_hbm, idx_d, ones_v, acc, ssem):
    cid = lax.axis_index("c")
    sid = lax.axis_index("s")
    _zero_acc(zeros_hbm, acc, sid)
    pltpu.sync_copy(ones_hbm, ones_v)
    plsc.subcore_barrier()
    rows_sub = (ER // NC) // NS
    base = cid * (ER // NC) + sid * rows_sub

    def step(b, carry):
        pltpu.sync_copy(dst_hbm.at[pl.ds(base + b * (IB * RPB), IB * RPB)],
                        idx_d)
        descs = [pltpu.async_copy(ones_v, acc.at[idx_d.at[j]], ssem, add=True)
                 for j in range(IB * RPB)]
        for d in descs:
            d.wait()
        return carry

    lax.fori_loop(0, rows_sub // (IB * RPB), step, 0)
    plsc.subcore_barrier()
    _write_out(acc, out_hbm, sid, cid)


_deg_call = pl.kernel(
    _deg_body,
    out_type=jax.ShapeDtypeStruct((NC * N, 8), jnp.float32),
    mesh=_MESH,
    compiler_params=_SC_PARAMS,
    scratch_types=[
        pltpu.VMEM((IB * RPB, EB), jnp.int32),
        pltpu.VMEM((EB, 8), jnp.float32),
        pltpu.VMEM_SHARED((N + PAD, 8), jnp.float32),
        pltpu.SemaphoreType.DMA,
    ],
)


def _make_agg(n_chunks):
    split_edges = n_chunks == 1
    tasks = max(n_chunks // NC, 1)
    rows_core = ER // NC if split_edges else ER
    rows_sub = rows_core // NS
    out_slots = max(n_chunks, 2)

    def body(srcoff_hbm, dst_hbm, table_hbm, zeros_hbm, out_hbm,
             idx_s, idx_d, rows_v, acc, gsem, ssem):
        cid = lax.axis_index("c")
        sid = lax.axis_index("s")
        for k in range(tasks):
            if split_edges:
                chunk = 0
                ebase = cid * rows_core
                slot = cid
            else:
                chunk = cid * tasks + k
                ebase = 0
                slot = chunk
            if split_edges:
                @pl.when(cid == 0)
                def _():
                    for kk in range(2):
                        ch = sid + NS * kk

                        @pl.when(ch < NZ)
                        def _(ch=ch):
                            pltpu.sync_copy(
                                table_hbm.at[pl.ds(ch * ZCH, ZCH)],
                                acc.at[pl.ds(ch * ZCH, ZCH)])

                @pl.when(cid == 1)
                def _():
                    _zero_acc(zeros_hbm, acc, sid)
            else:
                for kk in range(2):
                    ch = sid + NS * kk

                    @pl.when(ch < NZ)
                    def _(ch=ch):
                        pltpu.sync_copy(
                            table_hbm.at[pl.ds(chunk * N + ch * ZCH, ZCH)],
                            acc.at[pl.ds(ch * ZCH, ZCH)])
            plsc.subcore_barrier()
            base = chunk * ER + ebase + sid * rows_sub
            dbase = ebase + sid * rows_sub

            def step(b, carry):
                @pl.when(b > 0)
                def _():
                    for j in range(RPB):
                        pltpu.make_async_copy(
                            table_hbm.at[pl.ds(0, EB)],
                            rows_v.at[pl.ds(j * EB, EB)], ssem).wait()

                pltpu.sync_copy(
                    srcoff_hbm.at[pl.ds(base + b * (IB * RPB), IB * RPB)], idx_s)
                pltpu.sync_copy(
                    dst_hbm.at[pl.ds(dbase + b * (IB * RPB), IB * RPB)], idx_d)
                for b2 in range(IB):
                    gd = []
                    for j in range(RPB):
                        if b2 > 0:
                            pltpu.make_async_copy(
                                table_hbm.at[pl.ds(0, EB)],
                                rows_v.at[pl.ds(j * EB, EB)], ssem).wait()
                        gd.append(pltpu.async_copy(
                            table_hbm.at[idx_s.at[b2 * RPB + j]],
                            rows_v.at[pl.ds(j * EB, EB)], gsem))
                    for j in range(RPB):
                        gd[j].wait()
                        pltpu.async_copy(rows_v.at[pl.ds(j * EB, EB)],
                                         acc.at[idx_d.at[b2 * RPB + j]],
                                         ssem, add=True)
                return carry

            lax.fori_loop(0, rows_sub // (IB * RPB), step, 0)
            for j in range(RPB):
                pltpu.make_async_copy(table_hbm.at[pl.ds(0, EB)],
                                      rows_v.at[pl.ds(j * EB, EB)], ssem).wait()
            plsc.subcore_barrier()
            _write_out(acc, out_hbm, sid, slot)
            plsc.subcore_barrier()

    return pl.kernel(
        body,
        out_type=jax.ShapeDtypeStruct((out_slots * N, FC), jnp.float32),
        mesh=_MESH,
        compiler_params=_SC_PARAMS,
        scratch_types=[
            pltpu.VMEM((IB * RPB, EB), jnp.int32),
            pltpu.VMEM((IB * RPB, EB), jnp.int32),
            pltpu.VMEM((RPB * EB, FC), jnp.float32),
            pltpu.VMEM_SHARED((N + PAD, FC), jnp.float32),
            pltpu.SemaphoreType.DMA,
            pltpu.SemaphoreType.DMA,
        ],
    )


_agg1 = _make_agg(1)
_agg2 = _make_agg(2)
_agg3 = _make_agg(4)


def _prep_body(degp_ref, x_ref, w1_ref, hs1_ref):
    deg = degp_ref[0, :, 0:1] + degp_ref[1, :, 0:1] + 1.0
    dinv = lax.rsqrt(deg)
    h = jnp.dot(x_ref[...], w1_ref[...], preferred_element_type=jnp.float32)
    hs1_ref[0] = dinv * h


_prep_call = pl.pallas_call(
    _prep_body,
    grid=(NB,),
    in_specs=[
        pl.BlockSpec((NC, BN, 8), lambda i: (0, i, 0)),
        pl.BlockSpec((BN, 3), lambda i: (i, 0)),
        pl.BlockSpec((3, FC), lambda i: (0, 0)),
    ],
    out_specs=pl.BlockSpec((1, BN, FC), lambda i: (0, i, 0)),
    out_shape=jax.ShapeDtypeStruct((1, N, FC), jnp.float32),
)


def _layer_t(agg_ref, dinv_ref, b_ref, c, partial):
    a = (agg_ref[0] + agg_ref[1]) if partial else agg_ref[c]
    dinv = dinv_ref[...].reshape(BN, 1)
    return dinv * a + b_ref[c:c + 1, :]


def _norm_relu(t, s1_acc, s2_acc, gw_ref, gb_ref, ga_ref, c):
    mu = s1_acc[c:c + 1, :] * (1.0 / N)
    m2 = s2_acc[c:c + 1, :] * (1.0 / N)
    al = ga_ref[c:c + 1, :]
    var = m2 - (2.0 * al - al * al) * mu * mu
    u = t - al * mu
    y = gw_ref[c:c + 1, :] * u * lax.rsqrt(var + 1e-5) + gb_ref[c:c + 1, :]
    return jnp.maximum(y, 0.0)


def _stats_phase(agg_ref, dinv_ref, b_ref, s1_acc, s2_acc, C, partial, i):
    @pl.when(i == 0)
    def _():
        s1_acc[...] = jnp.zeros_like(s1_acc)
        s2_acc[...] = jnp.zeros_like(s2_acc)

    for c in range(C):
        t = _layer_t(agg_ref, dinv_ref, b_ref, c, partial)
        s1_acc[c:c + 1, :] += jnp.sum(t, axis=0, keepdims=True)
        s2_acc[c:c + 1, :] += jnp.sum(t * t, axis=0, keepdims=True)


def _make_layer(S, C, partial, C_out):
    def body(agg_ref, dinv_ref, b_ref, gw_ref, gb_ref, ga_ref,
             wn_ref, hsn_ref, s1_acc, s2_acc):
        p = pl.program_id(0)
        i = pl.program_id(1)

        @pl.when(p == 0)
        def _():
            _stats_phase(agg_ref, dinv_ref, b_ref, s1_acc, s2_acc,
                         C, partial, i)

        @pl.when(p == 1)
        def _():
            hn = jnp.zeros((BN, FC * C_out), jnp.float32)
            for c in range(C):
                t = _layer_t(agg_ref, dinv_ref, b_ref, c, partial)
                y = _norm_relu(t, s1_acc, s2_acc, gw_ref, gb_ref, ga_ref, c)
                hn = hn + jnp.dot(y, wn_ref[c * FC:(c + 1) * FC, :],
                                  preferred_element_type=jnp.float32)
            hsn = dinv_ref[...].reshape(BN, 1) * hn
            for k in range(C_out):
                hsn_ref[k] = hsn[:, k * FC:(k + 1) * FC]

    return pl.pallas_call(
        body,
        grid=(2, NB),
        in_specs=[
            pl.BlockSpec((S, BN, FC), lambda p, i: (0, i, 0)),
            pl.BlockSpec((1, 1, BN), lambda p, i: (i, 0, 0)),
            pl.BlockSpec((C, FC), lambda p, i: (0, 0)),
            pl.BlockSpec((C, FC), lambda p, i: (0, 0)),
            pl.BlockSpec((C, FC), lambda p, i: (0, 0)),
            pl.BlockSpec((C, FC), lambda p, i: (0, 0)),
            pl.BlockSpec((C * FC, C_out * FC), lambda p, i: (0, 0)),
        ],
        out_specs=pl.BlockSpec((C_out, BN, FC), lambda p, i: (0, i, 0)),
        out_shape=jax.ShapeDtypeStruct((C_out, N, FC), jnp.float32),
        scratch_shapes=[
            pltpu.VMEM((C, FC), jnp.float32),
            pltpu.VMEM((C, FC), jnp.float32),
        ],
    )


def _pool_body(agg_ref, dinv_ref, b_ref, gw_ref, gb_ref, ga_ref,
               batch_ref, linw_ref, linb_ref, out_ref,
               s1_acc, s2_acc, pool_acc, cnt_acc):
    p = pl.program_id(0)
    i = pl.program_id(1)

    @pl.when(p == 0)
    def _():
        _stats_phase(agg_ref, dinv_ref, b_ref, s1_acc, s2_acc,
                     4, False, i)

    @pl.when(p == 1)
    def _():
        @pl.when(i == 0)
        def _():
            pool_acc[...] = jnp.zeros_like(pool_acc)
            cnt_acc[...] = jnp.zeros_like(cnt_acc)

        gid = lax.broadcasted_iota(jnp.int32, (NG, 1), 0)
        mask = (gid == batch_ref[0]).astype(jnp.float32)
        cnt_acc[...] += jnp.sum(mask, axis=1, keepdims=True)
        for c in range(4):
            t = _layer_t(agg_ref, dinv_ref, b_ref, c, False)
            y = _norm_relu(t, s1_acc, s2_acc, gw_ref, gb_ref, ga_ref, c)
            pool_acc[:, c * FC:(c + 1) * FC] += jnp.dot(
                mask, y, preferred_element_type=jnp.float32)

        @pl.when(i == NB - 1)
        def _():
            pooled = pool_acc[...] / jnp.maximum(cnt_acc[...], 1.0)
            out_ref[...] = jnp.dot(pooled, linw_ref[...],
                                   preferred_element_type=jnp.float32) + linb_ref[...]


_pool_call = pl.pallas_call(
    _pool_body,
    grid=(2, NB),
    in_specs=[
        pl.BlockSpec((4, BN, FC), lambda p, i: (0, i, 0)),
        pl.BlockSpec((1, 1, BN), lambda p, i: (i, 0, 0)),
        pl.BlockSpec((4, FC), lambda p, i: (0, 0)),
        pl.BlockSpec((4, FC), lambda p, i: (0, 0)),
        pl.BlockSpec((4, FC), lambda p, i: (0, 0)),
        pl.BlockSpec((4, FC), lambda p, i: (0, 0)),
        pl.BlockSpec((1, 1, BN), lambda p, i: (i, 0, 0)),
        pl.BlockSpec((4 * FC, 3), lambda p, i: (0, 0)),
        pl.BlockSpec((1, 3), lambda p, i: (0, 0)),
    ],
    out_specs=pl.BlockSpec((NG, 3), lambda p, i: (0, 0)),
    out_shape=jax.ShapeDtypeStruct((NG, 3), jnp.float32),
    scratch_shapes=[
        pltpu.VMEM((4, FC), jnp.float32),
        pltpu.VMEM((4, FC), jnp.float32),
        pltpu.VMEM((NG, 4 * FC), jnp.float32),
        pltpu.VMEM((NG, 1), jnp.float32),
    ],
)

_layer1 = _make_layer(2, 1, True, 2)
_layer2 = _make_layer(2, 2, False, 4)


def kernel(x, edge_index, batch, W1, b1, g1w, g1b, g1a, W2, b2, g2w, g2b, g2a,
           W3, b3, g3w, g3b, g3a, linW, linb):
    pad = EP - E
    src = jnp.concatenate(
        [edge_index[0], jnp.arange(pad, dtype=jnp.int32) % 1024])
    dst2d = jnp.concatenate(
        [edge_index[1],
         N + (jnp.arange(pad, dtype=jnp.int32) % PAD)]).reshape(ER, EB)
    srcoff = (src[None, :]
              + (jnp.arange(4, dtype=jnp.int32) * N)[:, None]).reshape(4 * ER, EB)
    zeros_fc = jnp.zeros((ZCH, FC), jnp.float32)
    zeros_8 = jnp.zeros((ZCH, 8), jnp.float32)
    ones_8 = jnp.ones((EB, 8), jnp.float32)

    degp = _deg_call(dst2d, ones_8, zeros_8).reshape(NC, N, 8)
    dinv = lax.rsqrt(degp[0, :, 0] + degp[1, :, 0] + 1.0).reshape(NB, 1, BN)
    hs1 = _prep_call(degp, x, W1)

    agg1 = _agg1(srcoff, dst2d, hs1.reshape(N, FC), zeros_fc).reshape(2, N, FC)
    hs2 = _layer1(agg1, dinv, b1.reshape(1, FC),
                  g1w.reshape(1, FC), g1b.reshape(1, FC), g1a.reshape(1, FC), W2)

    agg2 = _agg2(srcoff, dst2d, hs2.reshape(2 * N, FC),
                 zeros_fc).reshape(2, N, FC)
    hs3 = _layer2(agg2, dinv, b2.reshape(2, FC),
                  g2w.reshape(2, FC), g2b.reshape(2, FC), g2a.reshape(2, FC), W3)

    agg3 = _agg3(srcoff, dst2d, hs3.reshape(4 * N, FC),
                 zeros_fc).reshape(4, N, FC)
    out = _pool_call(agg3, dinv, b3.reshape(4, FC),
                     g3w.reshape(4, FC), g3b.reshape(4, FC), g3a.reshape(4, FC),
                     batch.reshape(NB, 1, BN), linW, linb.reshape(1, 3))
    return out
